# baseline (speedup 1.0000x reference)
import jax
import jax.numpy as jnp
from jax.experimental import pallas as pl

def _rk(x_ref, o_ref):
    o_ref[...] = x_ref[:, :8, :128][None]

def kernel(inp, active_block_indices, bin_counts, W, b):
    xf = inp.reshape(1, 512, 49152)
    out = pl.pallas_call(
        _rk,
        grid=(16,),
        in_specs=[pl.BlockSpec((1, 32, 49152), lambda i: (0, i, 0))],
        out_specs=pl.BlockSpec((1, 1, 8, 128), lambda i: (0, i, 0, 0)),
        out_shape=jax.ShapeDtypeStruct((1, 16, 8, 128), jnp.float32),
    )(xf)
    return out
